# speedup vs baseline: 15.3100x; 15.3100x over previous
import functools

import jax
import jax.numpy as jnp
from jax.experimental import pallas as pl

EXT = 500
W = 2048


def _block(vp_ref, art_ref, aw_ref, gen_ref, attn_ref, out_ref, *, n_vocab):
    vb = pl.program_id(1)
    gen = gen_ref[0]
    attn = attn_ref[0]
    w = jnp.sum(attn * gen, axis=1, keepdims=True)
    ids = jax.lax.broadcasted_iota(jnp.int32, (1, W), 1) + vb * W
    mask = ids < n_vocab
    vp = vp_ref[0]
    acc = jnp.where(mask, vp * w, 0.0)
    k2 = attn * (1.0 - gen)
    n_agents = gen.shape[1]
    for a in range(n_agents):
        art_a = art_ref[0][:, a:a + 1]
        onehot = (art_a == ids).astype(jnp.float32)
        c_a = aw_ref[0][:, a, :] * k2[:, a:a + 1]
        acc = acc + jax.lax.dot(c_a, onehot,
                                preferred_element_type=jnp.float32)
    out_ref[0] = acc


def kernel(article, vocab_probs, generation_probs, agentwise_attn, agent_attn):
    bsz, n_agents, src_len = article.shape
    tgt_len, n_vocab = vocab_probs.shape[1], vocab_probs.shape[2]
    vx = n_vocab + EXT
    art_t = jnp.transpose(article, (0, 2, 1)).astype(jnp.int32)
    nb = pl.cdiv(vx, W)
    body = functools.partial(_block, n_vocab=n_vocab)
    out = pl.pallas_call(
        body,
        grid=(bsz, nb),
        in_specs=[
            pl.BlockSpec((1, tgt_len, W), lambda b, vb: (b, 0, vb)),
            pl.BlockSpec((1, src_len, n_agents), lambda b, vb: (b, 0, 0)),
            pl.BlockSpec((1, tgt_len, n_agents, src_len),
                         lambda b, vb: (b, 0, 0, 0)),
            pl.BlockSpec((1, tgt_len, n_agents), lambda b, vb: (b, 0, 0)),
            pl.BlockSpec((1, tgt_len, n_agents), lambda b, vb: (b, 0, 0)),
        ],
        out_specs=pl.BlockSpec((1, tgt_len, W), lambda b, vb: (b, 0, vb)),
        out_shape=jax.ShapeDtypeStruct((bsz, tgt_len, vx), jnp.float32),
    )(vocab_probs, art_t, agentwise_attn, generation_probs, agent_attn)
    return out
